# Initial kernel scaffold; baseline (speedup 1.0000x reference)
import jax
import jax.numpy as jnp
from jax.experimental import pallas as pl


def kernel(x, edge_index, edge_type, edge_weight, W):
    raise NotImplementedError("write your pallas kernel here")


# speedup vs baseline: 10.4636x; 10.4636x over previous
import functools

import jax
import jax.numpy as jnp
from jax import lax
from jax.experimental import pallas as pl
from jax.experimental.pallas import tpu as pltpu
from jax.experimental.pallas import tpu_sc as plsc

BATCH = 128


def _xw_table(x, W):
    N, FIN = x.shape
    R, FOUT, _ = W.shape
    BN = 2000
    nb = N // BN

    def body(x_ref, w_ref, o_ref):
        o_ref[0] = lax.dot_general(
            x_ref[...], w_ref[0], (((1,), (1,)), ((), ())),
            preferred_element_type=jnp.float32)

    return pl.pallas_call(
        body,
        grid=(nb, R),
        in_specs=[
            pl.BlockSpec((BN, FIN), lambda i, r: (i, 0)),
            pl.BlockSpec((1, FOUT, FIN), lambda i, r: (r, 0, 0)),
        ],
        out_specs=pl.BlockSpec((1, BN, FOUT), lambda i, r: (r, i, 0)),
        out_shape=jax.ShapeDtypeStruct((R, N, FOUT), jnp.float32),
    )(x, W)


def _materialize(*arrays):
    def body(*refs):
        n = len(refs) // 2
        for i in range(n):
            refs[n + i][...] = refs[i][...]

    return pl.pallas_call(
        body,
        out_shape=[jax.ShapeDtypeStruct(a.shape, a.dtype) for a in arrays],
    )(*arrays)


def _combine(partials, N):
    NC, _, F = partials.shape
    BN = 2000
    nb = N // BN

    def body(p_ref, o_ref):
        o_ref[...] = jnp.sum(p_ref[...], axis=0)

    return pl.pallas_call(
        body,
        grid=(nb,),
        in_specs=[pl.BlockSpec((NC, BN, F), lambda i: (0, i, 0))],
        out_specs=pl.BlockSpec((BN, F), lambda i: (i, 0)),
        out_shape=jax.ShapeDtypeStruct((N, F), jnp.float32),
    )(partials)


def _sc_gather_scale_scatter(xw_flat, idx3, dst3, ew3, y0, *, nc, ns, nl):
    nw = nc * ns
    nbatch = idx3.shape[1]
    Np, F = y0.shape
    stripe = Np // ns
    ncolv = F // nl

    mesh = plsc.VectorSubcoreMesh(core_axis_name="c", subcore_axis_name="s")

    @functools.partial(
        pl.kernel,
        mesh=mesh,
        out_type=jax.ShapeDtypeStruct((nc, Np, F), jnp.float32),
        scratch_types=[
            pltpu.VMEM((BATCH,), jnp.int32),
            pltpu.VMEM((BATCH,), jnp.int32),
            pltpu.VMEM((BATCH,), jnp.float32),
            pltpu.VMEM((BATCH, F), jnp.float32),
            pltpu.VMEM_SHARED((Np, F), jnp.float32),
            pltpu.SemaphoreType.DMA,
            pltpu.SemaphoreType.DMA,
        ],
        compiler_params=pltpu.CompilerParams(needs_layout_passes=False),
    )
    def sc_k(xw_hbm, idx_hbm, dst_hbm, ew_hbm, y0_hbm, out_hbm,
             idx_v, dst_v, ew_v, rows_v, acc, gsem, ssem):
        c = lax.axis_index("c")
        s = lax.axis_index("s")
        wid = s * nc + c

        pltpu.sync_copy(y0_hbm.at[pl.ds(s * stripe, stripe)],
                        acc.at[pl.ds(s * stripe, stripe)])
        plsc.subcore_barrier()

        def batch_step(j, carry):
            pltpu.sync_copy(idx_hbm.at[wid, j], idx_v)
            pltpu.sync_copy(dst_hbm.at[wid, j], dst_v)
            pltpu.sync_copy(ew_hbm.at[wid, j], ew_v)

            pltpu.async_copy(xw_hbm.at[idx_v], rows_v, gsem).wait()

            def scale(i, acc_):
                w = plsc.load_gather(ew_v, [jnp.full((nl,), i, jnp.int32)])
                for t in range(ncolv):
                    sl = (i, pl.ds(t * nl, nl))
                    rows_v[sl] = rows_v[sl] * w
                return acc_

            lax.fori_loop(0, BATCH, scale, 0, unroll=2)

            pltpu.async_copy(rows_v, acc.at[dst_v], ssem, add=True).wait()
            return carry

        lax.fori_loop(0, nbatch, batch_step, 0)
        plsc.subcore_barrier()

        pltpu.sync_copy(acc.at[pl.ds(s * stripe, stripe)],
                        out_hbm.at[c, pl.ds(s * stripe, stripe)])

    return sc_k(xw_flat, idx3, dst3, ew3, y0)


def kernel(x, edge_index, edge_type, edge_weight, W):
    N, FIN = x.shape
    R, FOUT, _ = W.shape
    E = edge_type.shape[0]

    nc, ns, nl = 2, 16, 16
    nw = nc * ns

    xw = _xw_table(x, W)
    xw_flat = xw.reshape(R * N, FOUT)

    per_w = -(-E // (nw * BATCH)) * BATCH
    e_pad = per_w * nw
    pad = e_pad - E

    dst = edge_index[0]
    src = edge_index[1]
    flat_idx = edge_type * N + src

    zi = jnp.zeros((pad,), jnp.int32)
    idx3 = jnp.concatenate([flat_idx, zi]).reshape(nw, per_w // BATCH, BATCH)
    dst3 = jnp.concatenate([dst, zi]).reshape(nw, per_w // BATCH, BATCH)
    ew3 = jnp.concatenate([edge_weight, jnp.zeros((pad,), jnp.float32)]
                          ).reshape(nw, per_w // BATCH, BATCH)

    stripe = -(-(-(-N // ns)) // 8) * 8
    n_pad = stripe * ns
    y0 = jnp.zeros((n_pad, FOUT), jnp.float32)

    idx3, dst3, ew3, y0 = _materialize(idx3, dst3, ew3, y0)
    xw_flat, idx3, dst3, ew3, y0 = lax.optimization_barrier(
        (xw_flat, idx3, dst3, ew3, y0))

    partials = _sc_gather_scale_scatter(xw_flat, idx3, dst3, ew3, y0,
                                        nc=nc, ns=ns, nl=nl)
    return _combine(partials, N)
